# Initial kernel scaffold; baseline (speedup 1.0000x reference)
import jax
import jax.numpy as jnp
from jax.experimental import pallas as pl


def kernel(scores):
    raise NotImplementedError("write your pallas kernel here")


# speedup vs baseline: 3.5147x; 3.5147x over previous
import jax
import jax.numpy as jnp
from jax import lax
from jax.experimental import pallas as pl
from jax.experimental.pallas import tpu as pltpu

_ROWS_PER_BLOCK = 8
_BIG = 1 << 30


def _topk_mask_kernel(x_ref, o_ref):
    x = x_ref[...]
    r, c = x.shape
    iota = lax.broadcasted_iota(jnp.int32, (r, c), 1)

    neg = jnp.float32(-jnp.inf)

    m1 = jnp.max(x, axis=1, keepdims=True)
    i1 = jnp.min(jnp.where(x == m1, iota, _BIG), axis=1, keepdims=True)
    x2 = jnp.where(iota == i1, neg, x)
    m2 = jnp.max(x2, axis=1, keepdims=True)
    i2 = jnp.min(jnp.where(x2 == m2, iota, _BIG), axis=1, keepdims=True)
    x3 = jnp.where(iota == i2, neg, x2)
    m3 = jnp.max(x3, axis=1, keepdims=True)
    i3 = jnp.min(jnp.where(x3 == m3, iota, _BIG), axis=1, keepdims=True)

    denom = jnp.maximum(m1 + m2 + m3, jnp.float32(1e-12))
    sel = (iota == i1) | (iota == i2) | (iota == i3)
    o_ref[...] = jnp.where(sel, x / denom, jnp.float32(0.0))


def kernel(scores):
    n, c = scores.shape
    grid = n // _ROWS_PER_BLOCK
    return pl.pallas_call(
        _topk_mask_kernel,
        grid=(grid,),
        in_specs=[pl.BlockSpec((_ROWS_PER_BLOCK, c), lambda i: (i, 0))],
        out_specs=pl.BlockSpec((_ROWS_PER_BLOCK, c), lambda i: (i, 0)),
        out_shape=jax.ShapeDtypeStruct((n, c), scores.dtype),
    )(scores)


# speedup vs baseline: 3.5206x; 1.0017x over previous
import jax
import jax.numpy as jnp
from jax import lax
from jax.experimental import pallas as pl
from jax.experimental.pallas import tpu as pltpu

_ROWS_PER_BLOCK = 8
_BIG = 1 << 30


def _topk_mask_kernel(x_ref, o_ref):
    x = x_ref[...]
    r, c = x.shape
    iota = lax.broadcasted_iota(jnp.int32, (r, c), 1)

    neg = jnp.float32(-jnp.inf)

    m1 = jnp.max(x, axis=1, keepdims=True)
    i1 = jnp.min(jnp.where(x == m1, iota, _BIG), axis=1, keepdims=True)
    x2 = jnp.where(iota == i1, neg, x)
    m2 = jnp.max(x2, axis=1, keepdims=True)
    i2 = jnp.min(jnp.where(x2 == m2, iota, _BIG), axis=1, keepdims=True)
    x3 = jnp.where(iota == i2, neg, x2)
    m3 = jnp.max(x3, axis=1, keepdims=True)
    i3 = jnp.min(jnp.where(x3 == m3, iota, _BIG), axis=1, keepdims=True)

    inv = jnp.float32(1.0) / jnp.maximum(m1 + m2 + m3, jnp.float32(1e-12))
    sel = (iota == i1) | (iota == i2) | (iota == i3)
    o_ref[...] = jnp.where(sel, x * inv, jnp.float32(0.0))


def kernel(scores):
    n, c = scores.shape
    grid = n // _ROWS_PER_BLOCK
    return pl.pallas_call(
        _topk_mask_kernel,
        grid=(grid,),
        in_specs=[pl.BlockSpec((_ROWS_PER_BLOCK, c), lambda i: (i, 0))],
        out_specs=pl.BlockSpec((_ROWS_PER_BLOCK, c), lambda i: (i, 0)),
        out_shape=jax.ShapeDtypeStruct((n, c), scores.dtype),
    )(scores)
